# baseline (speedup 1.0000x reference)
import jax
import jax.numpy as jnp
from jax import lax
from jax.experimental import pallas as pl
from jax.experimental.pallas import tpu as pltpu
from jax.experimental.pallas import tpu_sc as plsc

N = 16
P = 32768
S = 1024
NC = 2
NS = 16
L = 16
HALF = P // 2


def _fps_body(pts_ref, p0_ref, idx_out, spts_out,
              xv, yv, zv, dist, idxbuf, sptsbuf, rowi, prowi, p0v, shi):
    c = lax.axis_index("c")
    s = lax.axis_index("s")
    b = c * (NS // 2) + s // 2
    h = s % 2
    base = h * HALF

    iota = lax.iota(jnp.int32, L)
    lane0 = iota == 0
    lane3 = iota < 3
    colv = jnp.minimum(iota, 2)

    off = (b * 3) * P + base
    pltpu.sync_copy(pts_ref.at[pl.ds(off, HALF)], xv)
    pltpu.sync_copy(pts_ref.at[pl.ds(off + P, HALF)], yv)
    pltpu.sync_copy(pts_ref.at[pl.ds(off + 2 * P, HALF)], zv)

    inf_vec = jnp.full((L,), jnp.inf, jnp.float32)

    @plsc.parallel_loop(0, HALF, L)
    def _(i):
        dist[pl.ds(i, L)] = inf_vec

    def combine(m, jl, tout):
        jsplat = jnp.full((L,), jl, jnp.int32)
        cxv = plsc.load_gather(xv, [jsplat])
        cyv = plsc.load_gather(yv, [jsplat])
        czv = plsc.load_gather(zv, [jsplat])
        jgv = jnp.full((L,), jl + base, jnp.int32)
        m_iv = plsc.bitcast(jnp.full((L,), m, jnp.float32), jnp.int32)
        cx_iv = plsc.bitcast(cxv, jnp.int32)
        cy_iv = plsc.bitcast(cyv, jnp.int32)
        cz_iv = plsc.bitcast(czv, jnp.int32)
        row = jnp.where(iota == 0, m_iv,
              jnp.where(iota == 1, cx_iv,
              jnp.where(iota == 2, cy_iv,
              jnp.where(iota == 3, cz_iv, jgv))))
        rowi[...] = row
        pltpu.sync_copy(rowi, shi.at[pl.ds(s * L, L)])
        plsc.subcore_barrier()
        pltpu.sync_copy(shi.at[pl.ds((s ^ 1) * L, L)], prowi)
        plsc.subcore_barrier()
        pv = prowi[...]
        pm_iv = jnp.full((L,), pv[0], jnp.int32)
        pjv = jnp.full((L,), pv[4], jnp.int32)
        pcxv = plsc.bitcast(jnp.full((L,), pv[1], jnp.int32), jnp.float32)
        pcyv = plsc.bitcast(jnp.full((L,), pv[2], jnp.int32), jnp.float32)
        pczv = plsc.bitcast(jnp.full((L,), pv[3], jnp.int32), jnp.float32)
        takev = (pm_iv > m_iv) | ((pm_iv == m_iv) & (pjv < jgv))
        wjv = jnp.where(takev, pjv, jgv)
        wcxv = jnp.where(takev, pcxv, cxv)
        wcyv = jnp.where(takev, pcyv, cyv)
        wczv = jnp.where(takev, pczv, czv)
        toutv = jnp.full((L,), tout, jnp.int32)
        plsc.store_scatter(idxbuf, [toutv], wjv, mask=lane0)
        vals = jnp.where(iota == 0, wcxv, jnp.where(iota == 1, wcyv, wczv))
        plsc.store_scatter(sptsbuf, [toutv * 3 + colv], vals, mask=lane3)
        return wcxv, wcyv, wczv

    pltpu.sync_copy(p0_ref.at[pl.ds(b * L, L)], p0v)
    v0 = p0v[...]
    cx0 = jnp.full((L,), v0[0], jnp.float32)
    cy0 = jnp.full((L,), v0[1], jnp.float32)
    cz0 = jnp.full((L,), v0[2], jnp.float32)
    zero = jnp.zeros((L,), jnp.int32)
    plsc.store_scatter(idxbuf, [zero], zero, mask=lane0)
    vals0 = jnp.where(iota == 0, cx0, jnp.where(iota == 1, cy0, cz0))
    plsc.store_scatter(sptsbuf, [colv], vals0, mask=lane3)
    carry0 = (cx0, cy0, cz0)

    def step(t, carry):
        cxv, cyv, czv = carry
        bv0 = jnp.full((L,), -jnp.inf, jnp.float32)
        bj0 = jnp.zeros((L,), jnp.int32)

        def pass_body(i, cr):
            bv, bj = cr
            xs = xv[pl.ds(i, L)]
            ys = yv[pl.ds(i, L)]
            zs = zv[pl.ds(i, L)]
            dv = dist[pl.ds(i, L)]
            dx = xs - cxv
            dy = ys - cyv
            dz = zs - czv
            d = dx * dx + dy * dy + dz * dz
            nd = jnp.minimum(dv, d)
            dist[pl.ds(i, L)] = nd
            upd = nd > bv
            bv = jnp.where(upd, nd, bv)
            bj = jnp.where(upd, iota + i, bj)
            return bv, bj

        bv, bj = plsc.parallel_loop(0, HALF, L, unroll=16,
                                    carry=(bv0, bj0))(pass_body)
        m = jnp.max(bv)
        jl = jnp.min(jnp.where(bv == m, bj, jnp.int32(2**30)))
        return combine(m, jl, t + 1)

    lax.fori_loop(0, S - 1, step, carry0)

    @pl.when(h == 0)
    def _():
        pltpu.sync_copy(idxbuf, idx_out.at[pl.ds(b * S, S)])
        pltpu.sync_copy(sptsbuf, spts_out.at[pl.ds(b * (3 * S), 3 * S)])


_mesh = plsc.VectorSubcoreMesh(core_axis_name="c", subcore_axis_name="s",
                               num_cores=NC, num_subcores=NS)

_fps = pl.kernel(
    _fps_body,
    out_type=(jax.ShapeDtypeStruct((N * S,), jnp.int32),
              jax.ShapeDtypeStruct((N * S * 3,), jnp.float32)),
    mesh=_mesh,
    compiler_params=pltpu.CompilerParams(needs_layout_passes=False),
    scratch_types=[
        pltpu.VMEM((HALF,), jnp.float32),
        pltpu.VMEM((HALF,), jnp.float32),
        pltpu.VMEM((HALF,), jnp.float32),
        pltpu.VMEM((HALF,), jnp.float32),
        pltpu.VMEM((S,), jnp.int32),
        pltpu.VMEM((3 * S,), jnp.float32),
        pltpu.VMEM((L,), jnp.int32),
        pltpu.VMEM((L,), jnp.int32),
        pltpu.VMEM((L,), jnp.float32),
        pltpu.VMEM_SHARED((NS * L,), jnp.int32),
    ],
)


def kernel(points, nsamples, return_gathered):
    pts_t = jnp.transpose(points, (0, 2, 1))
    p0s = jnp.zeros((N, L), jnp.float32).at[:, :3].set(points[:, 0, :])
    idx, spts = _fps(pts_t.reshape(-1), p0s.reshape(-1))
    idx = idx.reshape(N, S)
    spts = spts.reshape(N, S, 3)
    spts = jnp.where(jnp.asarray(return_gathered) != 0, spts,
                     jnp.zeros_like(spts))
    return (idx, spts)


# speedup vs baseline: 1.2180x; 1.2180x over previous
import jax
import jax.numpy as jnp
from jax import lax
from jax.experimental import pallas as pl
from jax.experimental.pallas import tpu as pltpu
from jax.experimental.pallas import tpu_sc as plsc

N = 16
P = 32768
S = 1024
NC = 2
NS = 16
L = 16
HALF = P // 2


def _fps_body(pts_ref, p0_ref, idx_out, spts_out,
              xv, yv, zv, dist, idxbuf, sptsbuf, rowi, prowi, p0v, shi):
    c = lax.axis_index("c")
    s = lax.axis_index("s")
    b = c * (NS // 2) + s // 2
    h = s % 2
    base = h * HALF

    iota = lax.iota(jnp.int32, L)
    lane0 = iota == 0
    lane3 = iota < 3
    colv = jnp.minimum(iota, 2)

    off = (b * 3) * P + base
    pltpu.sync_copy(pts_ref.at[pl.ds(off, HALF)], xv)
    pltpu.sync_copy(pts_ref.at[pl.ds(off + P, HALF)], yv)
    pltpu.sync_copy(pts_ref.at[pl.ds(off + 2 * P, HALF)], zv)

    inf_vec = jnp.full((L,), jnp.inf, jnp.float32)

    @plsc.parallel_loop(0, HALF, L)
    def _(i):
        dist[pl.ds(i, L)] = inf_vec

    def combine(m, jl, tout, slot):
        jsplat = jnp.full((L,), jl, jnp.int32)
        cxv = plsc.load_gather(xv, [jsplat])
        cyv = plsc.load_gather(yv, [jsplat])
        czv = plsc.load_gather(zv, [jsplat])
        jgv = jnp.full((L,), jl + base, jnp.int32)
        m_iv = plsc.bitcast(jnp.full((L,), m, jnp.float32), jnp.int32)
        cx_iv = plsc.bitcast(cxv, jnp.int32)
        cy_iv = plsc.bitcast(cyv, jnp.int32)
        cz_iv = plsc.bitcast(czv, jnp.int32)
        row = jnp.where(iota == 0, m_iv,
              jnp.where(iota == 1, cx_iv,
              jnp.where(iota == 2, cy_iv,
              jnp.where(iota == 3, cz_iv, jgv))))
        rowi[...] = row
        sbase = slot * (NS * L)
        pltpu.sync_copy(rowi, shi.at[pl.ds(sbase + s * L, L)])
        plsc.subcore_barrier()
        pltpu.sync_copy(shi.at[pl.ds(sbase + (s ^ 1) * L, L)], prowi)
        pv = prowi[...]
        pm_iv = jnp.full((L,), pv[0], jnp.int32)
        pjv = jnp.full((L,), pv[4], jnp.int32)
        pcxv = plsc.bitcast(jnp.full((L,), pv[1], jnp.int32), jnp.float32)
        pcyv = plsc.bitcast(jnp.full((L,), pv[2], jnp.int32), jnp.float32)
        pczv = plsc.bitcast(jnp.full((L,), pv[3], jnp.int32), jnp.float32)
        takev = (pm_iv > m_iv) | ((pm_iv == m_iv) & (pjv < jgv))
        wjv = jnp.where(takev, pjv, jgv)
        wcxv = jnp.where(takev, pcxv, cxv)
        wcyv = jnp.where(takev, pcyv, cyv)
        wczv = jnp.where(takev, pczv, czv)
        toutv = jnp.full((L,), tout, jnp.int32)
        plsc.store_scatter(idxbuf, [toutv], wjv, mask=lane0)
        vals = jnp.where(iota == 0, wcxv, jnp.where(iota == 1, wcyv, wczv))
        plsc.store_scatter(sptsbuf, [toutv * 3 + colv], vals, mask=lane3)
        return wcxv, wcyv, wczv

    pltpu.sync_copy(p0_ref.at[pl.ds(b * L, L)], p0v)
    v0 = p0v[...]
    cx0 = jnp.full((L,), v0[0], jnp.float32)
    cy0 = jnp.full((L,), v0[1], jnp.float32)
    cz0 = jnp.full((L,), v0[2], jnp.float32)
    zero = jnp.zeros((L,), jnp.int32)
    plsc.store_scatter(idxbuf, [zero], zero, mask=lane0)
    vals0 = jnp.where(iota == 0, cx0, jnp.where(iota == 1, cy0, cz0))
    plsc.store_scatter(sptsbuf, [colv], vals0, mask=lane3)
    carry0 = (cx0, cy0, cz0)

    def step(t, carry):
        cxv, cyv, czv = carry
        bv0 = jnp.full((L,), -jnp.inf, jnp.float32)
        bj0 = jnp.zeros((L,), jnp.int32)

        def pass_body(i, cr):
            bv, bj = cr
            xs = xv[pl.ds(i, L)]
            ys = yv[pl.ds(i, L)]
            zs = zv[pl.ds(i, L)]
            dv = dist[pl.ds(i, L)]
            dx = xs - cxv
            dy = ys - cyv
            dz = zs - czv
            d = dx * dx + dy * dy + dz * dz
            nd = jnp.minimum(dv, d)
            dist[pl.ds(i, L)] = nd
            upd = nd > bv
            bv = jnp.where(upd, nd, bv)
            bj = jnp.where(upd, iota + i, bj)
            return bv, bj

        bv, bj = plsc.parallel_loop(0, HALF, L, unroll=8,
                                    carry=(bv0, bj0))(pass_body)
        m = jnp.max(bv)
        jl = jnp.min(jnp.where(bv == m, bj, jnp.int32(2**30)))
        return combine(m, jl, t + 1, t % 2)

    lax.fori_loop(0, S - 1, step, carry0)

    @pl.when(h == 0)
    def _():
        pltpu.sync_copy(idxbuf, idx_out.at[pl.ds(b * S, S)])
        pltpu.sync_copy(sptsbuf, spts_out.at[pl.ds(b * (3 * S), 3 * S)])


_mesh = plsc.VectorSubcoreMesh(core_axis_name="c", subcore_axis_name="s",
                               num_cores=NC, num_subcores=NS)

_fps = pl.kernel(
    _fps_body,
    out_type=(jax.ShapeDtypeStruct((N * S,), jnp.int32),
              jax.ShapeDtypeStruct((N * S * 3,), jnp.float32)),
    mesh=_mesh,
    compiler_params=pltpu.CompilerParams(needs_layout_passes=False),
    scratch_types=[
        pltpu.VMEM((HALF,), jnp.float32),
        pltpu.VMEM((HALF,), jnp.float32),
        pltpu.VMEM((HALF,), jnp.float32),
        pltpu.VMEM((HALF,), jnp.float32),
        pltpu.VMEM((S,), jnp.int32),
        pltpu.VMEM((3 * S,), jnp.float32),
        pltpu.VMEM((L,), jnp.int32),
        pltpu.VMEM((L,), jnp.int32),
        pltpu.VMEM((L,), jnp.float32),
        pltpu.VMEM_SHARED((2 * NS * L,), jnp.int32),
    ],
)


def kernel(points, nsamples, return_gathered):
    pts_t = jnp.transpose(points, (0, 2, 1))
    p0s = jnp.zeros((N, L), jnp.float32).at[:, :3].set(points[:, 0, :])
    idx, spts = _fps(pts_t.reshape(-1), p0s.reshape(-1))
    idx = idx.reshape(N, S)
    spts = spts.reshape(N, S, 3)
    spts = jnp.where(jnp.asarray(return_gathered) != 0, spts,
                     jnp.zeros_like(spts))
    return (idx, spts)
